# Initial kernel scaffold; baseline (speedup 1.0000x reference)
import jax
import jax.numpy as jnp
from jax.experimental import pallas as pl


def kernel(x, edge_index, W1, attn_l1, attn_r1, b1, W2, attn_l2, attn_r2, b2):
    raise NotImplementedError("write your pallas kernel here")


# speedup vs baseline: 24.4064x; 24.4064x over previous
import functools

import jax
import jax.numpy as jnp
from jax import lax
from jax.experimental import pallas as pl
from jax.experimental.pallas import tpu as pltpu
from jax.experimental.pallas import tpu_sc as plsc

N = 10000
E = 320000
D = 128
NP = 10240
L = 16
NC = 2
NS = 16
NW = NC * NS
BE = 128
NBLK = E // BE
ROWS_PER_TILE = NP // NS
RB = 256
NRB = NP // RB


def _dense_body(h_ref, w_ref, al_ref, ar_ref,
                feat_ref, el_ref, er_ref, m_ref, acc_ref):
    i = pl.program_id(0)
    feat = jnp.dot(h_ref[...], w_ref[...],
                   preferred_element_type=jnp.float32)
    feat_ref[...] = feat
    el = jnp.sum(feat * al_ref[...], axis=1, keepdims=True)
    er = jnp.sum(feat * ar_ref[...], axis=1, keepdims=True)
    el_ref[...] = el
    er_ref[...] = er

    @pl.when(i == 0)
    def _():
        acc_ref[...] = jnp.full((2, 128), -jnp.inf, jnp.float32)

    acc_ref[0:1, :] = jnp.maximum(acc_ref[0:1, :], jnp.max(el))
    acc_ref[1:2, :] = jnp.maximum(acc_ref[1:2, :], jnp.max(er))

    @pl.when(i == pl.num_programs(0) - 1)
    def _():
        s = acc_ref[0:1, :] + acc_ref[1:2, :]
        m_ref[...] = jnp.maximum(s, 0.2 * s)


def _dense_stage(h, W, al, ar):
    grid = (NRB,)
    return pl.pallas_call(
        _dense_body,
        grid=grid,
        in_specs=[
            pl.BlockSpec((RB, D), lambda i: (i, 0)),
            pl.BlockSpec((D, D), lambda i: (0, 0)),
            pl.BlockSpec((1, D), lambda i: (0, 0)),
            pl.BlockSpec((1, D), lambda i: (0, 0)),
        ],
        out_specs=[
            pl.BlockSpec((RB, D), lambda i: (i, 0)),
            pl.BlockSpec((RB, 1), lambda i: (i, 0)),
            pl.BlockSpec((RB, 1), lambda i: (i, 0)),
            pl.BlockSpec((1, 128), lambda i: (0, 0)),
        ],
        out_shape=[
            jax.ShapeDtypeStruct((NP, D), jnp.float32),
            jax.ShapeDtypeStruct((NP, 1), jnp.float32),
            jax.ShapeDtypeStruct((NP, 1), jnp.float32),
            jax.ShapeDtypeStruct((1, 128), jnp.float32),
        ],
        scratch_shapes=[pltpu.VMEM((2, 128), jnp.float32)],
    )(h, W.astype(jnp.float32), al.reshape(1, D), ar.reshape(1, D))


def _combine_body(num_ref, den_ref, b_ref, h_ref):
    nsum = num_ref[0] + num_ref[1]
    dsum = jnp.sum(den_ref[...], axis=1, keepdims=True)
    inv = jnp.where(dsum > 0.0, 1.0 / jnp.where(dsum > 0.0, dsum, 1.0), 0.0)
    h_ref[...] = jnp.maximum(nsum * inv + b_ref[...], 0.0)


def _combine_stage(numer, den_t, b):
    return pl.pallas_call(
        _combine_body,
        grid=(NRB,),
        in_specs=[
            pl.BlockSpec((2, RB, D), lambda i: (0, i, 0)),
            pl.BlockSpec((RB, NW), lambda i: (i, 0)),
            pl.BlockSpec((1, D), lambda i: (0, 0)),
        ],
        out_specs=pl.BlockSpec((RB, D), lambda i: (i, 0)),
        out_shape=jax.ShapeDtypeStruct((NP, D), jnp.float32),
    )(numer, den_t, b.reshape(1, D))


def _exp_neg(z):
    z = jnp.maximum(z, -80.0)
    t = z * 1.4426950408889634
    n = (t - 0.5).astype(jnp.int32)
    y = (t - n.astype(jnp.float32)) * 0.6931471805599453
    p = 1.0 + y * (1.0 + y * (0.5 + y * (
        0.16666666666666666 + y * (0.041666666666666664 + y * (
            0.008333333333333333 + y * 0.001388888888888889)))))
    pow2n = plsc.bitcast(lax.shift_left(n + 127, 23), jnp.float32)
    return pow2n * p


def _edge_body(feat_hbm, el_hbm, er_hbm, m_hbm, src_hbm, dst_hbm,
               numer_hbm, den_hbm,
               el_v, er_v, m_v, src_v, dst_v, rows_v, den_v, acc, sem):
    cid = lax.axis_index("c")
    sid = lax.axis_index("s")
    wid = sid * NC + cid

    pltpu.sync_copy(el_hbm, el_v)
    pltpu.sync_copy(er_hbm, er_v)
    pltpu.sync_copy(m_hbm, m_v)
    m16 = m_v[pl.ds(0, L)]

    zero16 = jnp.zeros((L,), jnp.float32)

    def zrow(r, carry):
        for c in range(D // L):
            rows_v[r, pl.ds(c * L, L)] = zero16
        return carry
    lax.fori_loop(0, BE, zrow, 0)

    def zden(i, carry):
        den_v[pl.ds(i * L, L)] = zero16
        return carry
    lax.fori_loop(0, NP // L, zden, 0)

    for j in range(ROWS_PER_TILE // BE):
        pltpu.sync_copy(
            rows_v, acc.at[pl.ds(sid * ROWS_PER_TILE + j * BE, BE), :])
    plsc.subcore_barrier()

    nblk = 78 + jnp.where(wid < NBLK - 78 * NW, 1, 0)

    def block_body(k, carry):
        b = wid + k * NW
        off = b * BE
        pltpu.sync_copy(src_hbm.at[pl.ds(off, BE)], src_v)
        pltpu.sync_copy(dst_hbm.at[pl.ds(off, BE)], dst_v)
        pltpu.async_copy(feat_hbm.at[src_v], rows_v, sem).wait()
        for g in range(BE // L):
            sv = src_v[pl.ds(g * L, L)]
            dv = dst_v[pl.ds(g * L, L)]
            elg = plsc.load_gather(el_v, [sv])
            erg = plsc.load_gather(er_v, [dv])
            s = elg + erg
            e = jnp.maximum(s, 0.2 * s)
            w = _exp_neg(e - m16)
            plsc.addupdate_scatter(den_v, [dv], w)
            for lane in range(L):
                ws = jnp.broadcast_to(w[lane], (L,))
                r = g * L + lane
                for c in range(D // L):
                    rows_v[r, pl.ds(c * L, L)] = (
                        rows_v[r, pl.ds(c * L, L)] * ws)
        pltpu.sync_copy(rows_v, acc.at[dst_v], add=True)
        return carry
    lax.fori_loop(0, nblk, block_body, 0)

    plsc.subcore_barrier()
    for j in range(ROWS_PER_TILE // BE):
        rowoff = sid * ROWS_PER_TILE + j * BE
        pltpu.sync_copy(acc.at[pl.ds(rowoff, BE), :],
                        numer_hbm.at[cid, pl.ds(rowoff, BE), :])
    pltpu.sync_copy(den_v, den_hbm.at[wid])


_edge_stage = functools.partial(
    pl.kernel,
    _edge_body,
    out_type=[
        jax.ShapeDtypeStruct((NC, NP, D), jnp.float32),
        jax.ShapeDtypeStruct((NW, NP), jnp.float32),
    ],
    mesh=plsc.VectorSubcoreMesh(core_axis_name="c", subcore_axis_name="s"),
    compiler_params=pltpu.CompilerParams(needs_layout_passes=False),
    scratch_types=[
        pltpu.VMEM((NP,), jnp.float32),
        pltpu.VMEM((NP,), jnp.float32),
        pltpu.VMEM((128,), jnp.float32),
        pltpu.VMEM((BE,), jnp.int32),
        pltpu.VMEM((BE,), jnp.int32),
        pltpu.VMEM((BE, D), jnp.float32),
        pltpu.VMEM((NP,), jnp.float32),
        pltpu.VMEM_SHARED((NP, D), jnp.float32),
        pltpu.SemaphoreType.DMA,
    ],
)()


def kernel(x, edge_index, W1, attn_l1, attn_r1, b1, W2, attn_l2, attn_r2, b2):
    src = edge_index[0].astype(jnp.int32)
    dst = edge_index[1].astype(jnp.int32)
    h0 = jnp.pad(x, ((0, NP - N), (0, 0)))

    feat1, el1, er1, m1 = _dense_stage(h0, W1, attn_l1, attn_r1)
    num1, den1 = _edge_stage(feat1, el1.reshape(NP), er1.reshape(NP),
                             m1.reshape(128), src, dst)
    h1 = _combine_stage(num1, den1.T, b1)

    feat2, el2, er2, m2 = _dense_stage(h1, W2, attn_l2, attn_r2)
    num2, den2 = _edge_stage(feat2, el2.reshape(NP), er2.reshape(NP),
                             m2.reshape(128), src, dst)
    h2 = _combine_stage(num2, den2.T, b2)
    return h2[:N]


# speedup vs baseline: 30.0678x; 1.2320x over previous
import functools

import jax
import jax.numpy as jnp
from jax import lax
from jax.experimental import pallas as pl
from jax.experimental.pallas import tpu as pltpu
from jax.experimental.pallas import tpu_sc as plsc

N = 10000
E = 320000
D = 128
NP = 10240
L = 16
NC = 2
NS = 16
NW = NC * NS
BE = 128
NBLK = E // BE
ROWS_PER_TILE = NP // NS
RB = 256
NRB = NP // RB


def _dense_body(h_ref, w_ref, al_ref, ar_ref,
                feat_ref, el_ref, er_ref, m_ref, acc_ref):
    i = pl.program_id(0)
    feat = jnp.dot(h_ref[...], w_ref[...],
                   preferred_element_type=jnp.float32)
    feat_ref[...] = feat
    el = jnp.sum(feat * al_ref[...], axis=1, keepdims=True)
    er = jnp.sum(feat * ar_ref[...], axis=1, keepdims=True)
    el_ref[...] = el
    er_ref[...] = er

    @pl.when(i == 0)
    def _():
        acc_ref[...] = jnp.full((2, 128), -jnp.inf, jnp.float32)

    acc_ref[0:1, :] = jnp.maximum(acc_ref[0:1, :], jnp.max(el))
    acc_ref[1:2, :] = jnp.maximum(acc_ref[1:2, :], jnp.max(er))

    @pl.when(i == pl.num_programs(0) - 1)
    def _():
        s = acc_ref[0:1, :] + acc_ref[1:2, :]
        m_ref[...] = jnp.maximum(s, 0.2 * s)


def _dense_stage(h, W, al, ar):
    grid = (NRB,)
    return pl.pallas_call(
        _dense_body,
        grid=grid,
        in_specs=[
            pl.BlockSpec((RB, D), lambda i: (i, 0)),
            pl.BlockSpec((D, D), lambda i: (0, 0)),
            pl.BlockSpec((1, D), lambda i: (0, 0)),
            pl.BlockSpec((1, D), lambda i: (0, 0)),
        ],
        out_specs=[
            pl.BlockSpec((RB, D), lambda i: (i, 0)),
            pl.BlockSpec((RB, 1), lambda i: (i, 0)),
            pl.BlockSpec((RB, 1), lambda i: (i, 0)),
            pl.BlockSpec((1, 128), lambda i: (0, 0)),
        ],
        out_shape=[
            jax.ShapeDtypeStruct((NP, D), jnp.float32),
            jax.ShapeDtypeStruct((NP, 1), jnp.float32),
            jax.ShapeDtypeStruct((NP, 1), jnp.float32),
            jax.ShapeDtypeStruct((1, 128), jnp.float32),
        ],
        scratch_shapes=[pltpu.VMEM((2, 128), jnp.float32)],
    )(h, W.astype(jnp.float32), al.reshape(1, D), ar.reshape(1, D))


def _combine_body(num_ref, den_ref, b_ref, h_ref):
    nsum = num_ref[0] + num_ref[1]
    dsum = jnp.sum(den_ref[...], axis=1, keepdims=True)
    inv = jnp.where(dsum > 0.0, 1.0 / jnp.where(dsum > 0.0, dsum, 1.0), 0.0)
    h_ref[...] = jnp.maximum(nsum * inv + b_ref[...], 0.0)


def _combine_stage(numer, den_t, b):
    return pl.pallas_call(
        _combine_body,
        grid=(NRB,),
        in_specs=[
            pl.BlockSpec((2, RB, D), lambda i: (0, i, 0)),
            pl.BlockSpec((RB, NW), lambda i: (i, 0)),
            pl.BlockSpec((1, D), lambda i: (0, 0)),
        ],
        out_specs=pl.BlockSpec((RB, D), lambda i: (i, 0)),
        out_shape=jax.ShapeDtypeStruct((NP, D), jnp.float32),
    )(numer, den_t, b.reshape(1, D))


def _exp_neg(z):
    z = jnp.maximum(z, -80.0)
    t = z * 1.4426950408889634
    n = (t - 0.5).astype(jnp.int32)
    y = (t - n.astype(jnp.float32)) * 0.6931471805599453
    p = 1.0 + y * (1.0 + y * (0.5 + y * (
        0.16666666666666666 + y * (0.041666666666666664 + y * (
            0.008333333333333333 + y * 0.001388888888888889)))))
    pow2n = plsc.bitcast(lax.shift_left(n + 127, 23), jnp.float32)
    return pow2n * p


def _edge_body(feat_hbm, el_hbm, er_hbm, m_hbm, ei_hbm,
               numer_hbm, den_hbm,
               m_v, idx0, idx1, rows0, rows1, elg0, elg1, erg0, erg1,
               den_v, acc, sem0, sem1):
    cid = lax.axis_index("c")
    sid = lax.axis_index("s")
    wid = sid * NC + cid

    pltpu.sync_copy(m_hbm, m_v)
    m16 = m_v[pl.ds(0, L)]

    zero16 = jnp.zeros((L,), jnp.float32)

    def zrow(r, carry):
        for c in range(D // L):
            rows0[r, pl.ds(c * L, L)] = zero16
        return carry
    lax.fori_loop(0, BE, zrow, 0)

    def zden(i, carry):
        den_v[pl.ds(i * L, L)] = zero16
        return carry
    lax.fori_loop(0, NP // L, zden, 0)

    for j in range(ROWS_PER_TILE // BE):
        pltpu.sync_copy(
            rows0, acc.at[pl.ds(sid * ROWS_PER_TILE + j * BE, BE), :])
    plsc.subcore_barrier()

    nblk = 78 + jnp.where(wid < NBLK - 78 * NW, 1, 0)

    def issue(idx_b, rows_b, elg_b, erg_b, sem_b, k):
        off = (wid + k * NW) * BE
        pltpu.sync_copy(ei_hbm.at[:, pl.ds(off, BE)], idx_b)
        pltpu.async_copy(feat_hbm.at[idx_b.at[0]], rows_b, sem_b)
        pltpu.async_copy(el_hbm.at[idx_b.at[0]], elg_b, sem_b)
        pltpu.async_copy(er_hbm.at[idx_b.at[1]], erg_b, sem_b)

    def process(idx_b, rows_b, elg_b, erg_b, sem_b):
        pltpu.make_async_copy(feat_hbm.at[idx_b.at[0]], rows_b, sem_b).wait()
        pltpu.make_async_copy(el_hbm.at[idx_b.at[0]], elg_b, sem_b).wait()
        pltpu.make_async_copy(er_hbm.at[idx_b.at[1]], erg_b, sem_b).wait()
        for g in range(BE // L):
            dv = idx_b[1, pl.ds(g * L, L)]
            elg = elg_b[pl.ds(g * L, L)]
            erg = erg_b[pl.ds(g * L, L)]
            s = elg + erg
            e = jnp.maximum(s, 0.2 * s)
            w = _exp_neg(e - m16)
            plsc.addupdate_scatter(den_v, [dv], w)
            for lane in range(L):
                ws = jnp.broadcast_to(w[lane], (L,))
                r = g * L + lane
                for c in range(D // L):
                    rows_b[r, pl.ds(c * L, L)] = (
                        rows_b[r, pl.ds(c * L, L)] * ws)
        pltpu.sync_copy(rows_b, acc.at[idx_b.at[1]], add=True)

    issue(idx0, rows0, elg0, erg0, sem0, 0)

    def pair_body(j, carry):
        k0 = 2 * j
        issue(idx1, rows1, elg1, erg1, sem1, k0 + 1)
        process(idx0, rows0, elg0, erg0, sem0)

        @pl.when(k0 + 2 < nblk)
        def _():
            issue(idx0, rows0, elg0, erg0, sem0, k0 + 2)
        process(idx1, rows1, elg1, erg1, sem1)
        return carry
    lax.fori_loop(0, 39, pair_body, 0)

    @pl.when(nblk > 78)
    def _():
        process(idx0, rows0, elg0, erg0, sem0)

    plsc.subcore_barrier()
    for j in range(ROWS_PER_TILE // BE):
        rowoff = sid * ROWS_PER_TILE + j * BE
        pltpu.sync_copy(acc.at[pl.ds(rowoff, BE), :],
                        numer_hbm.at[cid, pl.ds(rowoff, BE), :])
    pltpu.sync_copy(den_v, den_hbm.at[wid])


_edge_stage = functools.partial(
    pl.kernel,
    _edge_body,
    out_type=[
        jax.ShapeDtypeStruct((NC, NP, D), jnp.float32),
        jax.ShapeDtypeStruct((NW, NP), jnp.float32),
    ],
    mesh=plsc.VectorSubcoreMesh(core_axis_name="c", subcore_axis_name="s"),
    compiler_params=pltpu.CompilerParams(needs_layout_passes=False),
    scratch_types=[
        pltpu.VMEM((128,), jnp.float32),
        pltpu.VMEM((2, BE), jnp.int32),
        pltpu.VMEM((2, BE), jnp.int32),
        pltpu.VMEM((BE, D), jnp.float32),
        pltpu.VMEM((BE, D), jnp.float32),
        pltpu.VMEM((BE,), jnp.float32),
        pltpu.VMEM((BE,), jnp.float32),
        pltpu.VMEM((BE,), jnp.float32),
        pltpu.VMEM((BE,), jnp.float32),
        pltpu.VMEM((NP,), jnp.float32),
        pltpu.VMEM_SHARED((NP, D), jnp.float32),
        pltpu.SemaphoreType.DMA,
        pltpu.SemaphoreType.DMA,
    ],
)()


def kernel(x, edge_index, W1, attn_l1, attn_r1, b1, W2, attn_l2, attn_r2, b2):
    ei = edge_index.astype(jnp.int32)
    h0 = jnp.pad(x, ((0, NP - N), (0, 0)))

    feat1, el1, er1, m1 = _dense_stage(h0, W1, attn_l1, attn_r1)
    num1, den1 = _edge_stage(feat1, el1.reshape(NP), er1.reshape(NP),
                             m1.reshape(128), ei)
    h1 = _combine_stage(num1, den1.T, b1)

    feat2, el2, er2, m2 = _dense_stage(h1, W2, attn_l2, attn_r2)
    num2, den2 = _edge_stage(feat2, el2.reshape(NP), er2.reshape(NP),
                             m2.reshape(128), ei)
    h2 = _combine_stage(num2, den2.T, b2)
    return h2[:N]
